# baseline (speedup 1.0000x reference)
import functools

import jax
import jax.numpy as jnp
from jax.experimental import pallas as pl
from jax.experimental.pallas import tpu as pltpu


def _router_body(chan_emb_ref, wr1_ref, br1_ref, wr2_ref, br2_ref, w1f_ref,
                 gates_ref, w1sum_ref, *, R, E):
    hidden = jnp.maximum(
        jax.lax.dot_general(
            chan_emb_ref[...], wr1_ref[...],
            (((1,), (0,)), ((), ())), preferred_element_type=jnp.float32,
        ) + br1_ref[...],
        0.0,
    )
    logits = jax.lax.dot_general(
        hidden, wr2_ref[...],
        (((1,), (0,)), ((), ())), preferred_element_type=jnp.float32,
    ) + br2_ref[...]
    m = jnp.max(logits, axis=-1, keepdims=True)
    ex = jnp.exp(logits - m)
    gates = ex / jnp.sum(ex, axis=-1, keepdims=True)
    gates_ref[...] = jnp.concatenate(
        [jnp.broadcast_to(gates[:, e:e + 1], gates.shape[:1] + (R,))
         for e in range(E)], axis=1)
    w1sum_ref[...] = jnp.sum(w1f_ref[...], axis=0, keepdims=True)


def _moe_body(x_hbm, w1f_ref, w2f_ref, gx_ref, w1s_ref, o_ref, xbuf, sems,
              *, L, C, NCH, NBUF):
    k = pl.program_id(0)

    def cp(idx, slot):
        return pltpu.make_async_copy(
            x_hbm.at[pl.ds(idx * C, C)], xbuf.at[slot], sems.at[slot])

    @pl.when(k == 0)
    def _prime():
        cp(0, 0).start()
        cp(1, 1).start()

    @pl.when(k < NCH - 2)
    def _ahead():
        cp(k + 2, (k + 2) % NBUF).start()

    cp(k, k % NBUF).wait()

    for i in range(C):
        xb = xbuf[k % NBUF, i]
        s1 = jnp.sum(xb, axis=0, keepdims=True)
        s2 = jnp.sum(xb * xb, axis=0, keepdims=True)
        mean = s1 * (1.0 / L)
        var = (s2 - mean * s1) * (1.0 / (L - 1))
        std = jnp.sqrt(var) + 1e-6
        rstd = 1.0 / std
        g = jax.lax.dot_general(
            xb, w1f_ref[...],
            (((0,), (0,)), ((), ())), preferred_element_type=jnp.float32,
        )
        mean_c = jnp.transpose(mean)
        rstd_c = jnp.transpose(rstd)
        hg = (g - mean_c * w1s_ref[...]) * (rstd_c * gx_ref[...])
        out_t = jax.lax.dot_general(
            w2f_ref[...], hg,
            (((0,), (1,)), ((), ())), preferred_element_type=jnp.float32,
        )
        o_ref[i] = out_t * std + mean


def kernel(x, chan_emb, Wr1, br1, Wr2, br2, W1, W2):
    B, L, N = x.shape
    E, _, R = W1.shape
    O = W2.shape[2]
    ER = E * R

    w1f = jnp.transpose(W1, (1, 0, 2)).reshape(L, ER)
    w2f = W2.reshape(ER, O)

    gates_ex, w1sum = pl.pallas_call(
        functools.partial(_router_body, R=R, E=E),
        out_shape=(
            jax.ShapeDtypeStruct((N, ER), jnp.float32),
            jax.ShapeDtypeStruct((1, ER), jnp.float32),
        ),
    )(chan_emb, Wr1, br1.reshape(1, -1), Wr2, br2.reshape(1, -1), w1f)

    C = 4
    NBUF = 3
    NCH = B // C
    out = pl.pallas_call(
        functools.partial(_moe_body, L=L, C=C, NCH=NCH, NBUF=NBUF),
        grid=(NCH,),
        in_specs=[
            pl.BlockSpec(memory_space=pl.ANY),
            pl.BlockSpec((L, ER), lambda b: (0, 0)),
            pl.BlockSpec((ER, O), lambda b: (0, 0)),
            pl.BlockSpec((N, ER), lambda b: (0, 0)),
            pl.BlockSpec((1, ER), lambda b: (0, 0)),
        ],
        out_specs=pl.BlockSpec((C, O, N), lambda b: (b, 0, 0)),
        out_shape=jax.ShapeDtypeStruct((B, O, N), jnp.float32),
        scratch_shapes=[
            pltpu.VMEM((NBUF, C, L, N), jnp.float32),
            pltpu.SemaphoreType.DMA((NBUF,)),
        ],
    )(x, w1f, w2f, gates_ex, w1sum)
    return out
